# baseline (speedup 1.0000x reference)
import functools

import jax
import jax.numpy as jnp
from jax import lax
from jax.experimental import pallas as pl
from jax.experimental.pallas import tpu as pltpu
from jax.experimental.pallas import tpu_sc as plsc

N = 8192
M = 8192
P = 1024
NW = 32
PW = P // NW
RU = 4
LANES = 16

TR = 1024
GRID = (N - P) // TR
CC = 1024
NCC = M // CC


def _tc_kernel(a_ref, b_ref, s1_ref, cmin_ref, acc_ref):
    i = pl.program_id(0)

    a = a_ref[...]
    row_mins = []
    col_mins = []
    for c in range(NCC):
        dc = jnp.dot(a, b_ref[:, c * CC:(c + 1) * CC],
                     preferred_element_type=jnp.float32)
        row_mins.append(jnp.min(dc, axis=1))
        col_mins.append(jnp.min(dc, axis=0))
    row_min = row_mins[0]
    for rm in row_mins[1:]:
        row_min = jnp.minimum(row_min, rm)
    s1_part = jnp.sum(jnp.sqrt(jnp.maximum(row_min, 0.0)))
    col_min = jnp.concatenate(col_mins)[None, :]

    @pl.when(i == 0)
    def _init():
        acc_ref[...] = col_min
        s1_ref[0, 0] = s1_part

    @pl.when(i > 0)
    def _acc():
        acc_ref[...] = jnp.minimum(acc_ref[...], col_min)
        s1_ref[0, 0] = s1_ref[0, 0] + s1_part

    @pl.when(i == GRID - 1)
    def _finish():
        cmin_ref[...] = acc_ref[...]


def _sc_kernel(st_hbm, tgr_hbm, rmin_hbm, cmin_hbm,
               st_v, tgr_v, rmin_v, cmin_v):
    cid = lax.axis_index("c")
    sid = lax.axis_index("s")
    wid = sid * 2 + cid
    base = wid * PW

    pltpu.sync_copy(st_hbm, st_v)
    for comp in range(4):
        pltpu.sync_copy(tgr_hbm.at[comp, pl.ds(base, PW), :],
                        tgr_v.at[comp])

    big = jnp.full((LANES,), 3.0e38, jnp.float32)

    def col_init(jc, _):
        cmin_v[pl.ds(jc * LANES, LANES)] = big
        return 0

    lax.fori_loop(0, M // LANES, col_init, 0)

    def row_block(rb, _):
        r0 = rb * RU
        txs, tys, tzs, a2s, rmins = [], [], [], [], []
        for u in range(RU):
            txs.append(tgr_v[0, r0 + u, :])
            tys.append(tgr_v[1, r0 + u, :])
            tzs.append(tgr_v[2, r0 + u, :])
            a2s.append(tgr_v[3, r0 + u, :])
            rmins.append(big)

        def chunk(jc, carry):
            rm = list(carry)
            sl = pl.ds(jc * LANES, LANES)
            sx = st_v[0, sl]
            sy = st_v[1, sl]
            sz = st_v[2, sl]
            b2 = st_v[3, sl]
            cm = cmin_v[sl]
            for u in range(RU):
                d = ((txs[u] * sx + tys[u] * sy) + (tzs[u] * sz + b2)) \
                    + a2s[u]
                rm[u] = jnp.minimum(rm[u], d)
                cm = jnp.minimum(cm, d)
            cmin_v[sl] = cm
            return tuple(rm)

        rm = lax.fori_loop(0, M // LANES, chunk, tuple(rmins))
        for u in range(RU):
            rmin_v[r0 + u, :] = rm[u]
        return 0

    lax.fori_loop(0, PW // RU, row_block, 0)

    pltpu.sync_copy(rmin_v, rmin_hbm.at[pl.ds(base, PW), :])
    pltpu.sync_copy(cmin_v, cmin_hbm.at[wid])


def _merge_kernel(s1_ref, cmin_tc_ref, rmin_sc_ref, cmin_sc_ref, out_ref):
    rmin = jnp.maximum(jnp.min(rmin_sc_ref[...], axis=1), 0.0)
    s1 = s1_ref[0, 0] + jnp.sum(jnp.sqrt(rmin))
    cmin = jnp.minimum(cmin_tc_ref[...],
                       jnp.min(cmin_sc_ref[...], axis=0, keepdims=True))
    s2 = jnp.sum(jnp.sqrt(jnp.maximum(cmin, 0.0)))
    out_ref[0, 0] = (s1 / N + s2 / M) * 5.0


def _round_bf16(x):
    u = jax.lax.bitcast_convert_type(x, jnp.uint32)
    r = (u + jnp.uint32(0x7FFF) + ((u >> 16) & jnp.uint32(1))) \
        & jnp.uint32(0xFFFF0000)
    return jax.lax.bitcast_convert_type(r, jnp.float32)


def _hi_lo(x):
    xi = jax.lax.bitcast_convert_type(x, jnp.uint32)
    hi_f = jax.lax.bitcast_convert_type(xi & jnp.uint32(0xFFFF0000),
                                        jnp.float32)
    hi = hi_f.astype(jnp.bfloat16)
    lo = (x - hi_f).astype(jnp.bfloat16)
    return hi, lo


@jax.jit
def _chamfer(state_x, target):
    a2 = jnp.sum(target * target, axis=1)
    b2 = jnp.sum(state_x * state_x, axis=1)

    a2h, a2l = _hi_lo(a2)
    b2h, b2l = _hi_lo(b2)
    one = jnp.ones((N,), jnp.bfloat16)
    zero = jnp.zeros((N,), jnp.bfloat16)
    A = jnp.stack(
        [target[:, 0].astype(jnp.bfloat16),
         target[:, 1].astype(jnp.bfloat16),
         target[:, 2].astype(jnp.bfloat16),
         a2h, a2l, one, one, zero], axis=1)[P:]
    B = jnp.stack(
        [(-2.0 * state_x[:, 0]).astype(jnp.bfloat16),
         (-2.0 * state_x[:, 1]).astype(jnp.bfloat16),
         (-2.0 * state_x[:, 2]).astype(jnp.bfloat16),
         one, one, b2h, b2l, zero], axis=0)

    s1_tc, cmin_tc = pl.pallas_call(
        _tc_kernel,
        grid=(GRID,),
        in_specs=[
            pl.BlockSpec((TR, 8), lambda i: (i, 0)),
            pl.BlockSpec((8, M), lambda i: (0, 0)),
        ],
        out_specs=[
            pl.BlockSpec(memory_space=pltpu.SMEM),
            pl.BlockSpec((1, M), lambda i: (0, 0)),
        ],
        out_shape=[
            jax.ShapeDtypeStruct((1, 1), jnp.float32),
            jax.ShapeDtypeStruct((1, M), jnp.float32),
        ],
        scratch_shapes=[pltpu.VMEM((1, M), jnp.float32)],
    )(A, B)

    st4 = jnp.stack(
        [_round_bf16(-2.0 * state_x[:, 0]),
         _round_bf16(-2.0 * state_x[:, 1]),
         _round_bf16(-2.0 * state_x[:, 2]),
         b2], axis=0)
    tg4 = jnp.stack(
        [_round_bf16(target[:P, 0]),
         _round_bf16(target[:P, 1]),
         _round_bf16(target[:P, 2]),
         a2[:P]], axis=0)
    tgr = jnp.broadcast_to(tg4[:, :, None], (4, P, LANES))

    mesh = plsc.VectorSubcoreMesh(core_axis_name="c", subcore_axis_name="s")
    rmin_sc, cmin_sc = pl.kernel(
        _sc_kernel,
        mesh=mesh,
        out_type=[
            jax.ShapeDtypeStruct((P, LANES), jnp.float32),
            jax.ShapeDtypeStruct((NW, M), jnp.float32),
        ],
        scratch_types=[
            pltpu.VMEM((4, M), jnp.float32),
            pltpu.VMEM((4, PW, LANES), jnp.float32),
            pltpu.VMEM((PW, LANES), jnp.float32),
            pltpu.VMEM((M,), jnp.float32),
        ],
    )(st4, tgr)

    loss = pl.pallas_call(
        _merge_kernel,
        in_specs=[
            pl.BlockSpec(memory_space=pltpu.SMEM),
            pl.BlockSpec(memory_space=pltpu.VMEM),
            pl.BlockSpec(memory_space=pltpu.VMEM),
            pl.BlockSpec(memory_space=pltpu.VMEM),
        ],
        out_specs=pl.BlockSpec(memory_space=pltpu.SMEM),
        out_shape=jax.ShapeDtypeStruct((1, 1), jnp.float32),
    )(s1_tc, cmin_tc, rmin_sc, cmin_sc)
    return loss[0, 0]


def kernel(state_x, target):
    return _chamfer(state_x, target)


# speedup vs baseline: 1.9851x; 1.9851x over previous
import jax
import jax.numpy as jnp
from jax.experimental import pallas as pl
from jax.experimental.pallas import tpu as pltpu

N = 8192
M = 8192
TR = 4096
GRID = N // TR


CC = 2048
NCC = M // CC


def _chamfer_kernel(a_ref, b_ref, out_ref, cmin_ref, s1_ref):
    i = pl.program_id(0)

    a = a_ref[...]
    row_mins = []
    col_mins = []
    for c in range(NCC):
        dc = jnp.dot(a, b_ref[:, c * CC:(c + 1) * CC],
                     preferred_element_type=jnp.float32)
        row_mins.append(jnp.min(dc, axis=1))
        col_mins.append(jnp.min(dc, axis=0))
    row_min = row_mins[0]
    for rm in row_mins[1:]:
        row_min = jnp.minimum(row_min, rm)
    s1_part = jnp.sum(jnp.sqrt(jnp.maximum(row_min, 0.0)))
    col_min = jnp.concatenate(col_mins)[None, :]

    @pl.when(i == 0)
    def _init():
        cmin_ref[...] = col_min
        s1_ref[0, 0] = s1_part

    @pl.when(i > 0)
    def _acc():
        cmin_ref[...] = jnp.minimum(cmin_ref[...], col_min)
        s1_ref[0, 0] = s1_ref[0, 0] + s1_part

    @pl.when(i == GRID - 1)
    def _finish():
        dist2 = jnp.maximum(cmin_ref[...], 0.0)
        s2 = jnp.sum(jnp.sqrt(dist2))
        s1 = s1_ref[0, 0]
        out_ref[0, 0] = (s1 / N + s2 / M) * 5.0


def _hi_lo(x):
    xi = jax.lax.bitcast_convert_type(x, jnp.uint32)
    hi_f = jax.lax.bitcast_convert_type(xi & jnp.uint32(0xFFFF0000), jnp.float32)
    hi = hi_f.astype(jnp.bfloat16)
    lo = (x - hi_f).astype(jnp.bfloat16)
    return hi, lo


@jax.jit
def _chamfer(state_x, target):
    a2 = jnp.sum(target * target, axis=1)
    b2 = jnp.sum(state_x * state_x, axis=1)
    a2h, a2l = _hi_lo(a2)
    b2h, b2l = _hi_lo(b2)
    one = jnp.ones((N,), jnp.bfloat16)
    zero = jnp.zeros((N,), jnp.bfloat16)
    A = jnp.stack(
        [target[:, 0].astype(jnp.bfloat16),
         target[:, 1].astype(jnp.bfloat16),
         target[:, 2].astype(jnp.bfloat16),
         a2h, a2l, one, one, zero], axis=1)
    B = jnp.stack(
        [(-2.0 * state_x[:, 0]).astype(jnp.bfloat16),
         (-2.0 * state_x[:, 1]).astype(jnp.bfloat16),
         (-2.0 * state_x[:, 2]).astype(jnp.bfloat16),
         one, one, b2h, b2l, zero], axis=0)

    loss = pl.pallas_call(
        _chamfer_kernel,
        grid=(GRID,),
        in_specs=[
            pl.BlockSpec((TR, 8), lambda i: (i, 0)),
            pl.BlockSpec((8, M), lambda i: (0, 0)),
        ],
        out_specs=pl.BlockSpec(memory_space=pltpu.SMEM),
        out_shape=jax.ShapeDtypeStruct((1, 1), jnp.float32),
        scratch_shapes=[
            pltpu.VMEM((1, M), jnp.float32),
            pltpu.SMEM((1, 1), jnp.float32),
        ],
    )(A, B)
    return loss[0, 0]


def kernel(state_x, target):
    return _chamfer(state_x, target)
